# baseline (speedup 1.0000x reference)
import dataclasses
import functools

import jax
import jax.numpy as jnp
from jax import lax
from jax.experimental import pallas as pl
from jax.experimental.pallas import tpu as pltpu
from jax.experimental.pallas import tpu_sc as plsc

NC = 2
NS = 16
NW = NC * NS
CH = 80

_CP = pltpu.CompilerParams()
if "needs_layout_passes" in pltpu.CompilerParams.__dataclass_fields__:
  _CP = dataclasses.replace(_CP, needs_layout_passes=False)


def _seg_sum_call(n, d, e, with_cnt, K, CH=CH):
  mesh = plsc.VectorSubcoreMesh(core_axis_name="c", subcore_axis_name="s")
  epw = e // NW
  nch = epw // CH
  RC = CH
  nrc = n // RC
  assert epw % CH == 0 and n % RC == 0
  assert not with_cnt or CH % 16 == 0

  out_type = [jax.ShapeDtypeStruct((NC, n, d), jnp.float32)]
  scratch = ([pltpu.VMEM_SHARED((n, d), jnp.float32)]
             + [pltpu.VMEM((CH,), jnp.int32) for _ in range(K)]
             + [pltpu.VMEM((CH,), jnp.int32) for _ in range(K)]
             + [pltpu.VMEM((CH, d), jnp.float32) for _ in range(K)]
             + [pltpu.SemaphoreType.DMA for _ in range(3 * K)])
  if with_cnt:
    out_type.append(jax.ShapeDtypeStruct((NW, n), jnp.float32))
    scratch.append(pltpu.VMEM((n,), jnp.float32))

  def body(x_hbm, src_hbm, dst_hbm, *rest):
    if with_cnt:
      sum_out, cnt_out = rest[:2]
      rest = rest[2:]
    else:
      sum_out = rest[0]
      rest = rest[1:]
    sum_sp = rest[0]
    idxs = rest[1:1 + K]
    idxd = rest[1 + K:1 + 2 * K]
    rows = rest[1 + 2 * K:1 + 3 * K]
    semIS = rest[1 + 3 * K:1 + 4 * K]
    semID = rest[1 + 4 * K:1 + 5 * K]
    semG = rest[1 + 5 * K:1 + 6 * K]
    hist = rest[-1] if with_cnt else None
    c = lax.axis_index("c")
    s = lax.axis_index("s")
    wid = c * NS + s
    e0 = wid * epw

    @pl.loop(0, CH)
    def _(i):
      for j in range(d // 16):
        rows[0][i, pl.ds(16 * j, 16)] = jnp.zeros((16,), jnp.float32)
    if with_cnt:
      @pl.loop(0, n, step=16)
      def _(i):
        hist[pl.ds(i, 16)] = jnp.zeros((16,), jnp.float32)

    @pl.loop(s, nrc, step=NS)
    def _(j):
      pltpu.sync_copy(rows[0], sum_sp.at[pl.ds(j * RC, RC)])
    plsc.subcore_barrier()

    ones16 = jnp.full((16,), 1.0, jnp.float32)

    def fetch_idx(i, b):
      pltpu.async_copy(src_hbm.at[pl.ds(e0 + i * CH, CH)], idxs[b],
                       semIS[b])
      pltpu.async_copy(dst_hbm.at[pl.ds(e0 + i * CH, CH)], idxd[b],
                       semID[b])

    def launch_gather(b):
      pltpu.make_async_copy(src_hbm.at[pl.ds(0, CH)], idxs[b],
                            semIS[b]).wait()
      pltpu.async_copy(x_hbm.at[idxs[b]], rows[b], semG[b])

    def process(i, b, reissue):
      b2 = (b + K - 1) % K
      @pl.when(i + K - 1 < nch)
      def _():
        launch_gather(b2)
      pltpu.make_async_copy(x_hbm.at[idxs[b]], rows[b], semG[b]).wait()
      pltpu.make_async_copy(dst_hbm.at[pl.ds(0, CH)], idxd[b],
                            semID[b]).wait()
      pltpu.sync_copy(rows[b], sum_sp.at[idxd[b]], add=True)
      if with_cnt:
        for j in range(CH // 16):
          idx16 = idxd[b][pl.ds(16 * j, 16)]
          plsc.addupdate_scatter(hist, [idx16], ones16)
      if reissue:
        nxt = i + K
        @pl.when(nxt < nch)
        def _():
          fetch_idx(nxt, b)

    for b in range(K):
      fetch_idx(b, b)
    for b in range(K - 1):
      launch_gather(b)

    nfull = nch // K

    @pl.loop(0, nfull)
    def _(t):
      for b in range(K):
        process(t * K + b, b, True)

    for r in range(nch % K):
      process(nfull * K + r, r, False)

    plsc.subcore_barrier()

    @pl.loop(s, nrc, step=NS)
    def _(j):
      pltpu.sync_copy(sum_sp.at[pl.ds(j * RC, RC)], rows[0])
      pltpu.sync_copy(rows[0], sum_out.at[c, pl.ds(j * RC, RC)])
    if with_cnt:
      pltpu.sync_copy(hist, cnt_out.at[wid])

  return pl.kernel(
      body,
      out_type=tuple(out_type) if with_cnt else out_type[0],
      mesh=mesh,
      compiler_params=_CP,
      scratch_types=scratch,
  )


def _pair_gather_call(n, d, p, K=2):
  mesh = plsc.VectorSubcoreMesh(core_axis_name="c", subcore_axis_name="s")
  assert p % CH == 0
  nch = p // CH
  nt = -(-nch // NW)
  assert nt % K == 0

  scratch = ([pltpu.VMEM((CH,), jnp.int32) for _ in range(K)]
             + [pltpu.VMEM((CH,), jnp.int32) for _ in range(K)]
             + [pltpu.VMEM((CH, d), jnp.float32) for _ in range(K)]
             + [pltpu.VMEM((CH, d), jnp.float32) for _ in range(K)]
             + [pltpu.SemaphoreType.DMA for _ in range(4 * K)])

  def body(h_hbm, pa_hbm, pb_hbm, ha_out, hb_out, *rest):
    ia = rest[:K]
    ib = rest[K:2 * K]
    ra = rest[2 * K:3 * K]
    rb = rest[3 * K:4 * K]
    semIA = rest[4 * K:5 * K]
    semIB = rest[5 * K:6 * K]
    semA = rest[6 * K:7 * K]
    semB = rest[7 * K:8 * K]
    c = lax.axis_index("c")
    s = lax.axis_index("s")
    wid = c * NS + s

    def cid(t):
      return jnp.minimum(wid + t * NW, nch - 1)

    def fetch_idx(t, b):
      base = cid(t) * CH
      pltpu.async_copy(pa_hbm.at[pl.ds(base, CH)], ia[b], semIA[b])
      pltpu.async_copy(pb_hbm.at[pl.ds(base, CH)], ib[b], semIB[b])

    def launch_gather(b):
      pltpu.make_async_copy(pa_hbm.at[pl.ds(0, CH)], ia[b], semIA[b]).wait()
      pltpu.make_async_copy(pb_hbm.at[pl.ds(0, CH)], ib[b], semIB[b]).wait()
      pltpu.async_copy(h_hbm.at[ia[b]], ra[b], semA[b])
      pltpu.async_copy(h_hbm.at[ib[b]], rb[b], semB[b])

    def process(t, b, reissue):
      b2 = (b + 1) % K
      if reissue:
        @pl.when(t + 1 < nt)
        def _():
          launch_gather(b2)
      pltpu.make_async_copy(h_hbm.at[ia[b]], ra[b], semA[b]).wait()
      pltpu.make_async_copy(h_hbm.at[ib[b]], rb[b], semB[b]).wait()
      base = cid(t) * CH
      pltpu.sync_copy(ra[b], ha_out.at[pl.ds(base, CH)])
      pltpu.sync_copy(rb[b], hb_out.at[pl.ds(base, CH)])
      if reissue:
        @pl.when(t + K < nt)
        def _():
          fetch_idx(t + K, b)

    fetch_idx(0, 0)
    launch_gather(0)
    for b in range(1, K):
      fetch_idx(b, b)

    @pl.loop(0, nt // K)
    def _(tt):
      for b in range(K):
        process(tt * K + b, b, True)

  return pl.kernel(
      body,
      out_type=(jax.ShapeDtypeStruct((p, d), jnp.float32),
                jax.ShapeDtypeStruct((p, d), jnp.float32)),
      mesh=mesh,
      compiler_params=_CP,
      scratch_types=scratch,
  )


def _sage_layer_call(n, d, blk=2000):
  def body(sum_ref, cnt_ref, x_ref, wlt_ref, bl_ref, wrt_ref, o_ref):
    sums = sum_ref[0] + sum_ref[1]
    deg = jnp.sum(cnt_ref[...], axis=1, keepdims=True)
    mean = sums / jnp.maximum(deg, 1.0)
    h = jnp.dot(mean, wlt_ref[...], preferred_element_type=jnp.float32)
    h += jnp.dot(x_ref[...], wrt_ref[...], preferred_element_type=jnp.float32)
    o_ref[...] = jnp.maximum(h + bl_ref[...], 0.0)

  return pl.pallas_call(
      body,
      grid=(n // blk,),
      in_specs=[
          pl.BlockSpec((NC, blk, d), lambda i: (0, i, 0)),
          pl.BlockSpec((blk, NW), lambda i: (i, 0)),
          pl.BlockSpec((blk, d), lambda i: (i, 0)),
          pl.BlockSpec((d, d), lambda i: (0, 0)),
          pl.BlockSpec((1, d), lambda i: (0, 0)),
          pl.BlockSpec((d, d), lambda i: (0, 0)),
      ],
      out_specs=pl.BlockSpec((blk, d), lambda i: (i, 0)),
      out_shape=jax.ShapeDtypeStruct((n, d), jnp.float32),
  )


def _mlp_call(p, d, dh, blk=2000):
  def body(ha_ref, hb_ref, w3at_ref, w3bt_ref, b3_ref, w4t_ref, b4_ref, o_ref):
    hid = jnp.dot(ha_ref[...], w3at_ref[...], preferred_element_type=jnp.float32)
    hid += jnp.dot(hb_ref[...], w3bt_ref[...], preferred_element_type=jnp.float32)
    hid = jnp.maximum(hid + b3_ref[...], 0.0)
    o_ref[...] = jnp.dot(hid, w4t_ref[...],
                         preferred_element_type=jnp.float32) + b4_ref[...]

  return pl.pallas_call(
      body,
      grid=(p // blk,),
      in_specs=[
          pl.BlockSpec((blk, d), lambda i: (i, 0)),
          pl.BlockSpec((blk, d), lambda i: (i, 0)),
          pl.BlockSpec((d, dh), lambda i: (0, 0)),
          pl.BlockSpec((d, dh), lambda i: (0, 0)),
          pl.BlockSpec((1, dh), lambda i: (0, 0)),
          pl.BlockSpec((dh, 1), lambda i: (0, 0)),
          pl.BlockSpec((1, 1), lambda i: (0, 0)),
      ],
      out_specs=pl.BlockSpec((blk, 1), lambda i: (i, 0)),
      out_shape=jax.ShapeDtypeStruct((p, 1), jnp.float32),
  )


def kernel(x, edge_index, edge_pairs, Wl1, bl1, Wr1, Wl2, bl2, Wr2, W3, b3,
           W4, b4):
  n, d = x.shape
  e = edge_index.shape[1]
  p = edge_pairs.shape[0]
  dh = W3.shape[0]

  src = edge_index[0]
  dst = edge_index[1]

  seg1 = _seg_sum_call(n, d, e, with_cnt=True, K=3)
  seg2 = _seg_sum_call(n, d, e, with_cnt=False, K=4)
  layer = _sage_layer_call(n, d)
  pair_gather = _pair_gather_call(n, d, p // 2)
  mlp = _mlp_call(p // 2, d, dh)

  sum1, cnt = seg1(x, src, dst)
  cnt_t = cnt.T
  h1 = layer(sum1, cnt_t, x, Wl1.T, bl1.reshape(1, d), Wr1.T)
  sum2 = seg2(h1, src, dst)
  h2 = layer(sum2, cnt_t, h1, Wl2.T, bl2.reshape(1, d), Wr2.T)

  pa = edge_pairs[:, 0]
  pb = edge_pairs[:, 1]
  w3at, w3bt = W3[:, :d].T, W3[:, d:].T
  b3r, w4t, b4r = b3.reshape(1, dh), W4.T, b4.reshape(1, 1)
  ph = p // 2
  outs = []
  for sl in range(2):
    pa_s = lax.dynamic_slice_in_dim(pa, sl * ph, ph)
    pb_s = lax.dynamic_slice_in_dim(pb, sl * ph, ph)
    ha, hb = pair_gather(h2, pa_s, pb_s)
    outs.append(mlp(ha, hb, w3at, w3bt, b3r, w4t, b4r))
  return jnp.concatenate(outs, axis=0)


# speedup vs baseline: 1.1165x; 1.1165x over previous
import dataclasses
import functools

import jax
import jax.numpy as jnp
from jax import lax
from jax.experimental import pallas as pl
from jax.experimental.pallas import tpu as pltpu
from jax.experimental.pallas import tpu_sc as plsc

NC = 2
NS = 16
NW = NC * NS
CH = 80

_CP = pltpu.CompilerParams()
if "needs_layout_passes" in pltpu.CompilerParams.__dataclass_fields__:
  _CP = dataclasses.replace(_CP, needs_layout_passes=False)


def _seg_sum_call(n, d, e, with_cnt, K, CH=CH):
  mesh = plsc.VectorSubcoreMesh(core_axis_name="c", subcore_axis_name="s")
  epw = e // NW
  nch = epw // CH
  RC = CH
  nrc = n // RC
  assert epw % CH == 0 and n % RC == 0
  assert not with_cnt or CH % 16 == 0

  out_type = [jax.ShapeDtypeStruct((NC, n, d), jnp.float32)]
  scratch = ([pltpu.VMEM_SHARED((n, d), jnp.float32),
              pltpu.VMEM((epw,), jnp.int32)]
             + [pltpu.VMEM((CH,), jnp.int32) for _ in range(K)]
             + [pltpu.VMEM((CH, d), jnp.float32) for _ in range(K)]
             + [pltpu.SemaphoreType.DMA for _ in range(2 * K)])
  if with_cnt:
    out_type.append(jax.ShapeDtypeStruct((NW, n), jnp.float32))
    scratch.append(pltpu.VMEM((n,), jnp.float32))

  def body(x_hbm, src_hbm, dst_hbm, *rest):
    if with_cnt:
      sum_out, cnt_out = rest[:2]
      rest = rest[2:]
    else:
      sum_out = rest[0]
      rest = rest[1:]
    sum_sp, idx_s = rest[:2]
    idxd = rest[2:2 + K]
    rows = rest[2 + K:2 + 2 * K]
    semI = rest[2 + 2 * K:2 + 3 * K]
    semG = rest[2 + 3 * K:2 + 4 * K]
    hist = rest[-1] if with_cnt else None
    c = lax.axis_index("c")
    s = lax.axis_index("s")
    wid = c * NS + s
    e0 = wid * epw

    @pl.loop(0, CH)
    def _(i):
      for j in range(d // 16):
        rows[0][i, pl.ds(16 * j, 16)] = jnp.zeros((16,), jnp.float32)
    if with_cnt:
      @pl.loop(0, n, step=16)
      def _(i):
        hist[pl.ds(i, 16)] = jnp.zeros((16,), jnp.float32)

    pltpu.sync_copy(src_hbm.at[pl.ds(e0, epw)], idx_s)

    @pl.loop(s, nrc, step=NS)
    def _(j):
      pltpu.sync_copy(rows[0], sum_sp.at[pl.ds(j * RC, RC)])
    plsc.subcore_barrier()

    ones16 = jnp.full((16,), 1.0, jnp.float32)

    def fetch(i, b):
      pltpu.async_copy(dst_hbm.at[pl.ds(e0 + i * CH, CH)], idxd[b], semI[b])
      pltpu.async_copy(x_hbm.at[idx_s.at[pl.ds(i * CH, CH)]], rows[b],
                       semG[b])

    def process(i, b, reissue):
      pltpu.make_async_copy(dst_hbm.at[pl.ds(0, CH)], idxd[b],
                            semI[b]).wait()
      pltpu.make_async_copy(x_hbm.at[idx_s.at[pl.ds(0, CH)]], rows[b],
                            semG[b]).wait()
      pltpu.sync_copy(rows[b], sum_sp.at[idxd[b]], add=True)
      if with_cnt:
        for j in range(CH // 16):
          idx16 = idxd[b][pl.ds(16 * j, 16)]
          plsc.addupdate_scatter(hist, [idx16], ones16)
      if reissue:
        nxt = i + K
        @pl.when(nxt < nch)
        def _():
          fetch(nxt, b)

    for b in range(K):
      fetch(b, b)

    nfull = nch // K

    @pl.loop(0, nfull)
    def _(t):
      for b in range(K):
        process(t * K + b, b, True)

    for r in range(nch % K):
      process(nfull * K + r, r, False)

    plsc.subcore_barrier()

    @pl.loop(s, nrc, step=NS)
    def _(j):
      pltpu.sync_copy(sum_sp.at[pl.ds(j * RC, RC)], rows[0])
      pltpu.sync_copy(rows[0], sum_out.at[c, pl.ds(j * RC, RC)])
    if with_cnt:
      pltpu.sync_copy(hist, cnt_out.at[wid])

  return pl.kernel(
      body,
      out_type=tuple(out_type) if with_cnt else out_type[0],
      mesh=mesh,
      compiler_params=_CP,
      scratch_types=scratch,
  )


def _pair_gather_call(n, d, p, K=2):
  mesh = plsc.VectorSubcoreMesh(core_axis_name="c", subcore_axis_name="s")
  assert p % CH == 0
  nch = p // CH
  nt = -(-nch // NW)
  assert nt % K == 0

  scratch = ([pltpu.VMEM((CH,), jnp.int32) for _ in range(K)]
             + [pltpu.VMEM((CH,), jnp.int32) for _ in range(K)]
             + [pltpu.VMEM((CH, d), jnp.float32) for _ in range(K)]
             + [pltpu.VMEM((CH, d), jnp.float32) for _ in range(K)]
             + [pltpu.SemaphoreType.DMA for _ in range(4 * K)])

  def body(h_hbm, pa_hbm, pb_hbm, ha_out, hb_out, *rest):
    ia = rest[:K]
    ib = rest[K:2 * K]
    ra = rest[2 * K:3 * K]
    rb = rest[3 * K:4 * K]
    semIA = rest[4 * K:5 * K]
    semIB = rest[5 * K:6 * K]
    semA = rest[6 * K:7 * K]
    semB = rest[7 * K:8 * K]
    c = lax.axis_index("c")
    s = lax.axis_index("s")
    wid = c * NS + s

    def cid(t):
      return jnp.minimum(wid + t * NW, nch - 1)

    def fetch_idx(t, b):
      base = cid(t) * CH
      pltpu.async_copy(pa_hbm.at[pl.ds(base, CH)], ia[b], semIA[b])
      pltpu.async_copy(pb_hbm.at[pl.ds(base, CH)], ib[b], semIB[b])

    def launch_gather(b):
      pltpu.make_async_copy(pa_hbm.at[pl.ds(0, CH)], ia[b], semIA[b]).wait()
      pltpu.make_async_copy(pb_hbm.at[pl.ds(0, CH)], ib[b], semIB[b]).wait()
      pltpu.async_copy(h_hbm.at[ia[b]], ra[b], semA[b])
      pltpu.async_copy(h_hbm.at[ib[b]], rb[b], semB[b])

    def process(t, b, reissue):
      b2 = (b + 1) % K
      if reissue:
        @pl.when(t + 1 < nt)
        def _():
          launch_gather(b2)
      pltpu.make_async_copy(h_hbm.at[ia[b]], ra[b], semA[b]).wait()
      pltpu.make_async_copy(h_hbm.at[ib[b]], rb[b], semB[b]).wait()
      base = cid(t) * CH
      pltpu.sync_copy(ra[b], ha_out.at[pl.ds(base, CH)])
      pltpu.sync_copy(rb[b], hb_out.at[pl.ds(base, CH)])
      if reissue:
        @pl.when(t + K < nt)
        def _():
          fetch_idx(t + K, b)

    fetch_idx(0, 0)
    launch_gather(0)
    for b in range(1, K):
      fetch_idx(b, b)

    @pl.loop(0, nt // K)
    def _(tt):
      for b in range(K):
        process(tt * K + b, b, True)

  return pl.kernel(
      body,
      out_type=(jax.ShapeDtypeStruct((p, d), jnp.float32),
                jax.ShapeDtypeStruct((p, d), jnp.float32)),
      mesh=mesh,
      compiler_params=_CP,
      scratch_types=scratch,
  )


def _sage_layer_call(n, d, blk=2000):
  def body(sum_ref, cnt_ref, x_ref, wlt_ref, bl_ref, wrt_ref, o_ref):
    sums = sum_ref[0] + sum_ref[1]
    deg = jnp.sum(cnt_ref[...], axis=1, keepdims=True)
    mean = sums / jnp.maximum(deg, 1.0)
    h = jnp.dot(mean, wlt_ref[...], preferred_element_type=jnp.float32)
    h += jnp.dot(x_ref[...], wrt_ref[...], preferred_element_type=jnp.float32)
    o_ref[...] = jnp.maximum(h + bl_ref[...], 0.0)

  return pl.pallas_call(
      body,
      grid=(n // blk,),
      in_specs=[
          pl.BlockSpec((NC, blk, d), lambda i: (0, i, 0)),
          pl.BlockSpec((blk, NW), lambda i: (i, 0)),
          pl.BlockSpec((blk, d), lambda i: (i, 0)),
          pl.BlockSpec((d, d), lambda i: (0, 0)),
          pl.BlockSpec((1, d), lambda i: (0, 0)),
          pl.BlockSpec((d, d), lambda i: (0, 0)),
      ],
      out_specs=pl.BlockSpec((blk, d), lambda i: (i, 0)),
      out_shape=jax.ShapeDtypeStruct((n, d), jnp.float32),
  )


def _mlp_call(p, d, dh, blk=2000):
  def body(ha_ref, hb_ref, w3at_ref, w3bt_ref, b3_ref, w4t_ref, b4_ref, o_ref):
    hid = jnp.dot(ha_ref[...], w3at_ref[...], preferred_element_type=jnp.float32)
    hid += jnp.dot(hb_ref[...], w3bt_ref[...], preferred_element_type=jnp.float32)
    hid = jnp.maximum(hid + b3_ref[...], 0.0)
    o_ref[...] = jnp.dot(hid, w4t_ref[...],
                         preferred_element_type=jnp.float32) + b4_ref[...]

  return pl.pallas_call(
      body,
      grid=(p // blk,),
      in_specs=[
          pl.BlockSpec((blk, d), lambda i: (i, 0)),
          pl.BlockSpec((blk, d), lambda i: (i, 0)),
          pl.BlockSpec((d, dh), lambda i: (0, 0)),
          pl.BlockSpec((d, dh), lambda i: (0, 0)),
          pl.BlockSpec((1, dh), lambda i: (0, 0)),
          pl.BlockSpec((dh, 1), lambda i: (0, 0)),
          pl.BlockSpec((1, 1), lambda i: (0, 0)),
      ],
      out_specs=pl.BlockSpec((blk, 1), lambda i: (i, 0)),
      out_shape=jax.ShapeDtypeStruct((p, 1), jnp.float32),
  )


def kernel(x, edge_index, edge_pairs, Wl1, bl1, Wr1, Wl2, bl2, Wr2, W3, b3,
           W4, b4):
  n, d = x.shape
  e = edge_index.shape[1]
  p = edge_pairs.shape[0]
  dh = W3.shape[0]

  src = edge_index[0]
  dst = edge_index[1]

  seg1 = _seg_sum_call(n, d, e, with_cnt=True, K=2)
  seg2 = _seg_sum_call(n, d, e, with_cnt=False, K=3)
  layer = _sage_layer_call(n, d)
  pair_gather = _pair_gather_call(n, d, p // 2)
  mlp = _mlp_call(p // 2, d, dh)

  sum1, cnt = seg1(x, src, dst)
  cnt_t = cnt.T
  h1 = layer(sum1, cnt_t, x, Wl1.T, bl1.reshape(1, d), Wr1.T)
  sum2 = seg2(h1, src, dst)
  h2 = layer(sum2, cnt_t, h1, Wl2.T, bl2.reshape(1, d), Wr2.T)

  pa = edge_pairs[:, 0]
  pb = edge_pairs[:, 1]
  w3at, w3bt = W3[:, :d].T, W3[:, d:].T
  b3r, w4t, b4r = b3.reshape(1, dh), W4.T, b4.reshape(1, 1)
  ph = p // 2
  outs = []
  for sl in range(2):
    pa_s = lax.dynamic_slice_in_dim(pa, sl * ph, ph)
    pb_s = lax.dynamic_slice_in_dim(pb, sl * ph, ph)
    ha, hb = pair_gather(h2, pa_s, pb_s)
    outs.append(mlp(ha, hb, w3at, w3bt, b3r, w4t, b4r))
  return jnp.concatenate(outs, axis=0)
